# baseline (speedup 1.0000x reference)
import functools

import jax
import jax.numpy as jnp
from jax import lax
from jax.experimental import pallas as pl
from jax.experimental.pallas import tpu as pltpu
from jax.experimental.pallas import tpu_sc as plsc

_N_WORKERS = 32
_SPLITS = (104, 96)
_W = 128


def _sc_embed(inputs_flat, token_table, pos_table, *, s, d):
    n = inputs_flat.shape[0]
    n_per_w = n // _N_WORKERS
    chunk = s
    nchunks = n_per_w // chunk
    nbuf = 4
    lag = 2

    mesh = plsc.VectorSubcoreMesh(core_axis_name="c", subcore_axis_name="s")

    @functools.partial(
        pl.kernel,
        mesh=mesh,
        compiler_params=pltpu.CompilerParams(use_tc_tiling_on_sc=False),
        out_type=jax.ShapeDtypeStruct((n, _W), jnp.float32),
        scratch_types=[
            pltpu.VMEM((s, d), jnp.float32),
            pltpu.VMEM((n_per_w,), jnp.int32),
            pltpu.VMEM((nbuf, chunk, d), jnp.float32),
            pltpu.SemaphoreType.DMA,
            pltpu.SemaphoreType.DMA,
            pltpu.SemaphoreType.DMA,
            pltpu.SemaphoreType.DMA,
            pltpu.SemaphoreType.DMA,
            pltpu.SemaphoreType.DMA,
            pltpu.SemaphoreType.DMA,
            pltpu.SemaphoreType.DMA,
            pltpu.SemaphoreType.DMA,
        ],
    )
    def k(idx_hbm, tok_hbm, pos_hbm, out_hbm, pos_v, idx_v, rows_v,
          ssem, gsem0, gsem1, gsem2, gsem3, wsem0, wsem1, wsem2, wsem3):
        wid = lax.axis_index("s") * 2 + lax.axis_index("c")
        base = wid * n_per_w
        gsems = (gsem0, gsem1, gsem2, gsem3)
        wsems = (wsem0, wsem1, wsem2, wsem3)

        pltpu.async_copy(pos_hbm, pos_v, ssem).wait()
        pltpu.async_copy(idx_hbm.at[pl.ds(base, n_per_w)], idx_v, ssem).wait()

        def gather_parts(g, b):
            off = g * chunk
            parts = []
            lo = 0
            for w in _SPLITS:
                parts.append((
                    tok_hbm.at[idx_v.at[pl.ds(off + lo, w)]],
                    rows_v.at[b].at[pl.ds(lo, w)],
                ))
                lo += w
            return parts

        def fire_gather(g, b):
            for src, dst in gather_parts(g, b):
                pltpu.async_copy(src, dst, gsems[b])

        def wait_gather(g, b):
            for src, dst in gather_parts(g, b):
                pltpu.make_async_copy(src, dst, gsems[b]).wait()

        def add_pos(b):
            @pl.loop(0, chunk, step=8)
            def _(r0):
                for dr in range(8):
                    for c in range(0, d, 16):
                        slc = (pl.ds(r0 + dr, 1), pl.ds(c, 16))
                        plsc.addupdate(
                            rows_v.at[b].at[*slc], pos_v.at[*slc][...]
                        )

        def fire_write(g, b):
            pltpu.async_copy(
                rows_v.at[b],
                out_hbm.at[pl.ds(base + g * chunk, chunk), pl.ds(0, d)],
                wsems[b])

        def wait_write(g, b):
            pltpu.make_async_copy(
                rows_v.at[b],
                out_hbm.at[pl.ds(base + g * chunk, chunk), pl.ds(0, d)],
                wsems[b]).wait()

        fire_gather(0, 0)
        fire_gather(1, 1)
        for g in range(lag):
            wait_gather(g, g)
            add_pos(g)
            fire_write(g, g)
            fire_gather(g + lag, g + lag)
        for g in range(lag, nbuf):
            wait_gather(g, g)
            add_pos(g)
            fire_write(g, g)
            wait_write(g - lag, (g + lag) % nbuf)
            fire_gather(g + lag, (g + lag) % nbuf)

        @pl.loop(nbuf, nchunks - nbuf, step=nbuf)
        def _(g0):
            for b in range(nbuf):
                g = g0 + b
                wait_gather(g, b)
                add_pos(b)
                fire_write(g, b)
                wait_write(g - lag, (b + lag) % nbuf)
                fire_gather(g + lag, (b + lag) % nbuf)

        for b in range(nbuf):
            g = nchunks - nbuf + b
            wait_gather(g, b)
            add_pos(b)
            fire_write(g, b)
            if g + lag < nchunks:
                wait_write(g - lag, (b + lag) % nbuf)
                fire_gather(g + lag, (b + lag) % nbuf)
        for b in range(nbuf):
            wait_write(nchunks - nbuf + b, b)

    return k(inputs_flat, token_table, pos_table)


@jax.jit
def kernel(inputs, token_table, pos_table):
    b, s = inputs.shape
    d = token_table.shape[1]
    idx = inputs.reshape(b * s).astype(jnp.int32)
    out = _sc_embed(idx, token_table, pos_table, s=s, d=d)
    return out[:, :d].reshape(b, s, d)
